# baseline (speedup 1.0000x reference)
import functools

import jax
import jax.numpy as jnp
from jax import lax
from jax.experimental import pallas as pl
from jax.experimental.pallas import tpu as pltpu
from jax.experimental.pallas import tpu_sc as plsc

_NC = 2
_NS = 16
_NW = _NC * _NS
_IDXG = 128
_CHUNK = 512


@functools.lru_cache(maxsize=None)
def _make_sc_gather(v, d, n):
    gpc = _CHUNK // _IDXG
    rows_per_w = n // _NW
    nchunk = rows_per_w // _CHUNK
    assert rows_per_w * _NW == n and nchunk * _CHUNK == rows_per_w

    mesh = plsc.VectorSubcoreMesh(core_axis_name="c", subcore_axis_name="s")

    @functools.partial(
        pl.kernel,
        out_type=jax.ShapeDtypeStruct((n // 2, 2 * d), jnp.float32),
        mesh=mesh,
        scratch_types=[
            pltpu.VMEM((gpc, _IDXG), jnp.int32),
            pltpu.VMEM((_CHUNK, d), jnp.float32),
            pltpu.SemaphoreType.DMA,
        ],
        compiler_params=pltpu.CompilerParams(use_tc_tiling_on_sc=False),
    )
    def gather(table_hbm, idx_hbm, out_hbm, idx_v, rows_v, sem):
        wid = lax.axis_index("s") * _NC + lax.axis_index("c")

        def chunk(i, carry):
            g0 = (wid * nchunk + i) * gpc
            pltpu.sync_copy(idx_hbm.at[pl.ds(g0, gpc)], idx_v)
            cps = [
                pltpu.async_copy(
                    table_hbm.at[idx_v.at[j]],
                    rows_v.at[pl.ds(j * _IDXG, _IDXG)],
                    sem,
                )
                for j in range(gpc)
            ]
            for c in cps:
                c.wait()
            r0 = g0 * _IDXG // 2
            pltpu.sync_copy(
                rows_v.at[pl.ds(0, _CHUNK // 2)],
                out_hbm.at[pl.ds(r0, _CHUNK // 2), pl.ds(0, d)],
            )
            pltpu.sync_copy(
                rows_v.at[pl.ds(_CHUNK // 2, _CHUNK // 2)],
                out_hbm.at[pl.ds(r0, _CHUNK // 2), pl.ds(d, d)],
            )
            return carry

        lax.fori_loop(0, nchunk, chunk, 0)

    return gather


def _dense_t_body(x_ref, wpec_ref, w2t_ref, b_ref, g_ref, beta_ref, out_ref):
    bl, d2 = x_ref.shape
    dm = d2 // 2
    x = x_ref[...]
    w2t = w2t_ref[...]
    h_t = lax.dot_general(
        w2t, x, (((1,), (1,)), ((), ())),
        preferred_element_type=jnp.float32,
    )
    pqall = jnp.dot(w2t, wpec_ref[...], preferred_element_type=jnp.float32)
    sel = (
        lax.broadcasted_iota(jnp.int32, pqall.shape, 1) == pl.program_id(0)
    ).astype(jnp.float32)
    pqcol = jnp.sum(pqall * sel, axis=1, keepdims=True)
    h_t = h_t + pqcol + b_ref[...]
    h3 = h_t.reshape(2, dm, bl)
    m = jnp.mean(h3, axis=1, keepdims=True)
    hc = h3 - m
    v = jnp.mean(hc * hc, axis=1, keepdims=True)
    hn = (hc * lax.rsqrt(v + 1e-5)).reshape(d2, bl)
    hn = hn * g_ref[...] + beta_ref[...]
    out_ref[...] = (hn * jax.nn.sigmoid(hn)).reshape(2, dm, bl)


def _dense_t_body_c(x_ref, wpec_ref, w2t_ref, b_ref, g_ref, beta_ref,
                    carry_ref, out_ref):
    del carry_ref
    _dense_t_body(x_ref, wpec_ref, w2t_ref, b_ref, g_ref, beta_ref, out_ref)


def kernel(input_ids, wte, wpe, W, b, gamma, beta):
    B, S = input_ids.shape
    V, D = wte.shape
    N = B * S
    N2 = N // 2
    D2 = 2 * D

    ids_t = input_ids.T.astype(jnp.int32)
    h1 = ids_t[0::2].reshape(N2)
    h2 = ids_t[1::2].reshape(N2)
    half = _CHUNK // 2
    idx = (
        jnp.stack([h1.reshape(-1, half), h2.reshape(-1, half)], axis=1)
        .reshape(N // _IDXG, _IDXG)
    )
    z = jnp.zeros((D, D), jnp.float32)
    w2t = jnp.block([[W, z], [z, W]])
    bcol = jnp.concatenate([b, b]).reshape(D2, 1)
    gcol = jnp.concatenate([gamma, gamma]).reshape(D2, 1)
    betacol = jnp.concatenate([beta, beta]).reshape(D2, 1)
    wpec = jnp.transpose(wpe[:S].reshape(S // 2, D2))

    P = 10
    sp = S // 2 // P
    irows = N // _IDXG // P
    sc_gather = _make_sc_gather(V, D, N // P)
    x2s = [sc_gather(wte, idx[irows * p:irows * (p + 1)]) for p in range(P)]

    out_t = None
    for p in range(P):
        specs = [
            pl.BlockSpec((B, D2), lambda i: (i, 0)),
            pl.BlockSpec((D2, sp), lambda i: (0, 0)),
            pl.BlockSpec((D2, D2), lambda i: (0, 0)),
            pl.BlockSpec((D2, 1), lambda i: (0, 0)),
            pl.BlockSpec((D2, 1), lambda i: (0, 0)),
            pl.BlockSpec((D2, 1), lambda i: (0, 0)),
        ]
        args = [x2s[p], wpec[:, sp * p:sp * (p + 1)], w2t, bcol, gcol, betacol]
        out_spec = pl.BlockSpec((2, D, B), lambda i, pp=p: (sp * pp + i, 0, 0))
        if p == 0:
            out_t = pl.pallas_call(
                _dense_t_body,
                grid=(sp,),
                in_specs=specs,
                out_specs=out_spec,
                out_shape=jax.ShapeDtypeStruct((S, D, B), jnp.float32),
            )(*args)
        else:
            out_t = pl.pallas_call(
                _dense_t_body_c,
                grid=(sp,),
                in_specs=specs + [pl.BlockSpec(memory_space=pl.ANY)],
                out_specs=out_spec,
                out_shape=jax.ShapeDtypeStruct((S, D, B), jnp.float32),
                input_output_aliases={6: 0},
            )(*args, out_t)
    return jnp.transpose(out_t, (2, 0, 1))


# speedup vs baseline: 1.0086x; 1.0086x over previous
import functools

import jax
import jax.numpy as jnp
from jax import lax
from jax.experimental import pallas as pl
from jax.experimental.pallas import tpu as pltpu
from jax.experimental.pallas import tpu_sc as plsc

_NC = 2
_NS = 16
_NW = _NC * _NS
_IDXG = 128
_CHUNK = 512


@functools.lru_cache(maxsize=None)
def _make_sc_gather(v, d, n):
    gpc = _CHUNK // _IDXG
    rows_per_w = n // _NW
    nchunk = rows_per_w // _CHUNK
    assert rows_per_w * _NW == n and nchunk * _CHUNK == rows_per_w

    mesh = plsc.VectorSubcoreMesh(core_axis_name="c", subcore_axis_name="s")

    @functools.partial(
        pl.kernel,
        out_type=jax.ShapeDtypeStruct((n // 2, 2 * d), jnp.float32),
        mesh=mesh,
        scratch_types=[
            pltpu.VMEM((gpc, _IDXG), jnp.int32),
            pltpu.VMEM((_CHUNK, d), jnp.float32),
            pltpu.SemaphoreType.DMA,
        ],
        compiler_params=pltpu.CompilerParams(use_tc_tiling_on_sc=False),
    )
    def gather(table_hbm, idx_hbm, out_hbm, idx_v, rows_v, sem):
        wid = lax.axis_index("s") * _NC + lax.axis_index("c")

        def chunk(i, carry):
            g0 = (wid * nchunk + i) * gpc
            pltpu.sync_copy(idx_hbm.at[pl.ds(g0, gpc)], idx_v)
            cps = [
                pltpu.async_copy(
                    table_hbm.at[idx_v.at[j]],
                    rows_v.at[pl.ds(j * _IDXG, _IDXG)],
                    sem,
                )
                for j in range(gpc)
            ]
            for c in cps:
                c.wait()
            r0 = g0 * _IDXG // 2
            pltpu.sync_copy(
                rows_v.at[pl.ds(0, _CHUNK // 2)],
                out_hbm.at[pl.ds(r0, _CHUNK // 2), pl.ds(0, d)],
            )
            pltpu.sync_copy(
                rows_v.at[pl.ds(_CHUNK // 2, _CHUNK // 2)],
                out_hbm.at[pl.ds(r0, _CHUNK // 2), pl.ds(d, d)],
            )
            return carry

        lax.fori_loop(0, nchunk, chunk, 0)

    return gather


def _dense_t_body(x_ref, wpec_ref, w2t_ref, b_ref, g_ref, beta_ref, out_ref):
    bl, d2 = x_ref.shape
    dm = d2 // 2
    x = x_ref[...]
    w2t = w2t_ref[...]
    h_t = lax.dot_general(
        w2t, x, (((1,), (1,)), ((), ())),
        preferred_element_type=jnp.float32,
    )
    pqall = jnp.dot(w2t, wpec_ref[...], preferred_element_type=jnp.float32)
    sel = (
        lax.broadcasted_iota(jnp.int32, pqall.shape, 1) == pl.program_id(0)
    ).astype(jnp.float32)
    pqcol = jnp.sum(pqall * sel, axis=1, keepdims=True)
    h_t = h_t + pqcol + b_ref[...]
    h3 = h_t.reshape(2, dm, bl)
    m = jnp.mean(h3, axis=1, keepdims=True)
    hc = h3 - m
    v = jnp.mean(hc * hc, axis=1, keepdims=True)
    hn = (hc * lax.rsqrt(v + 1e-5)).reshape(d2, bl)
    hn = hn * g_ref[...] + beta_ref[...]
    out_ref[...] = (hn * jax.nn.sigmoid(hn)).reshape(2, dm, bl)


def _dense_t_body_c(x_ref, wpec_ref, w2t_ref, b_ref, g_ref, beta_ref,
                    carry_ref, out_ref):
    del carry_ref
    _dense_t_body(x_ref, wpec_ref, w2t_ref, b_ref, g_ref, beta_ref, out_ref)


def kernel(input_ids, wte, wpe, W, b, gamma, beta):
    B, S = input_ids.shape
    V, D = wte.shape
    N = B * S
    N2 = N // 2
    D2 = 2 * D

    ids_t = input_ids.T.astype(jnp.int32)
    h1 = ids_t[0::2].reshape(N2)
    h2 = ids_t[1::2].reshape(N2)
    half = _CHUNK // 2
    idx = (
        jnp.stack([h1.reshape(-1, half), h2.reshape(-1, half)], axis=1)
        .reshape(N // _IDXG, _IDXG)
    )
    z = jnp.zeros((D, D), jnp.float32)
    w2t = jnp.block([[W, z], [z, W]])
    bcol = jnp.concatenate([b, b]).reshape(D2, 1)
    gcol = jnp.concatenate([gamma, gamma]).reshape(D2, 1)
    betacol = jnp.concatenate([beta, beta]).reshape(D2, 1)
    wpec = jnp.transpose(wpe[:S].reshape(S // 2, D2))

    P = 5
    sp = S // 2 // P
    irows = N // _IDXG // P
    sc_gather = _make_sc_gather(V, D, N // P)
    x2s = [sc_gather(wte, idx[irows * p:irows * (p + 1)]) for p in range(P)]

    out_t = None
    for p in range(P):
        specs = [
            pl.BlockSpec((B, D2), lambda i: (i, 0)),
            pl.BlockSpec((D2, sp), lambda i: (0, 0)),
            pl.BlockSpec((D2, D2), lambda i: (0, 0)),
            pl.BlockSpec((D2, 1), lambda i: (0, 0)),
            pl.BlockSpec((D2, 1), lambda i: (0, 0)),
            pl.BlockSpec((D2, 1), lambda i: (0, 0)),
        ]
        args = [x2s[p], wpec[:, sp * p:sp * (p + 1)], w2t, bcol, gcol, betacol]
        out_spec = pl.BlockSpec((2, D, B), lambda i, pp=p: (sp * pp + i, 0, 0))
        if p == 0:
            out_t = pl.pallas_call(
                _dense_t_body,
                grid=(sp,),
                in_specs=specs,
                out_specs=out_spec,
                out_shape=jax.ShapeDtypeStruct((S, D, B), jnp.float32),
            )(*args)
        else:
            out_t = pl.pallas_call(
                _dense_t_body_c,
                grid=(sp,),
                in_specs=specs + [pl.BlockSpec(memory_space=pl.ANY)],
                out_specs=out_spec,
                out_shape=jax.ShapeDtypeStruct((S, D, B), jnp.float32),
                input_output_aliases={6: 0},
            )(*args, out_t)
    return jnp.transpose(out_t, (2, 0, 1))
